# baseline (speedup 1.0000x reference)
import functools

import jax
import jax.numpy as jnp
from jax import lax
from jax.experimental import pallas as pl
from jax.experimental.pallas import tpu as pltpu
from jax.experimental.pallas import tpu_sc as plsc

_N = 4096
_K = 32
_LANES = 16
_NC, _NS = 2, 16
_NW = _NC * _NS
_CHUNK = _N * _K // _NW

_ROWS = 512


def _sc_gather_body(t_hbm, idx_hbm, out_hbm, table_v, idx_v, out_v):
    wid = lax.axis_index("s") * _NC + lax.axis_index("c")
    base = wid * _CHUNK
    pltpu.sync_copy(t_hbm, table_v)
    pltpu.sync_copy(idx_hbm.at[pl.ds(base, _CHUNK)], idx_v)

    @pl.loop(0, _CHUNK // _LANES)
    def _(i):
        iv = idx_v[pl.ds(i * _LANES, _LANES)]
        valid = iv >= jnp.zeros((_LANES,), jnp.int32)
        safe = jnp.maximum(iv, jnp.zeros((_LANES,), jnp.int32))
        g = plsc.load_gather(table_v, [safe])
        out_v[pl.ds(i * _LANES, _LANES)] = jnp.where(
            valid, g, jnp.zeros((_LANES,), jnp.float32))

    pltpu.sync_copy(out_v, out_hbm.at[pl.ds(base, _CHUNK)])


@functools.cache
def _sc_gather():
    return pl.kernel(
        _sc_gather_body,
        out_type=jax.ShapeDtypeStruct((_N * _K,), jnp.float32),
        mesh=plsc.VectorSubcoreMesh(
            core_axis_name="c", subcore_axis_name="s",
            num_cores=_NC, num_subcores=_NS),
        scratch_types=[
            pltpu.VMEM((_N,), jnp.float32),
            pltpu.VMEM((_CHUNK,), jnp.int32),
            pltpu.VMEM((_CHUNK,), jnp.float32),
        ],
        compiler_params=pltpu.CompilerParams(needs_layout_passes=False),
    )


def _tc_outer_body(t_ref, n_ref, coef_ref, tgt_ref):
    nb = n_ref[...]
    tb = t_ref[...]
    b = jnp.concatenate([nb] * _K, axis=1)
    a = jnp.concatenate(
        [jnp.broadcast_to(nb[:, k:k + 1], (_ROWS, _K)) for k in range(_K)],
        axis=1)
    coef_ref[...] = a * b
    tgt_ref[...] = tb * nb


_tc_outer = pl.pallas_call(
    _tc_outer_body,
    grid=(_N // _ROWS,),
    in_specs=[
        pl.BlockSpec((_ROWS, 1), lambda i: (i, 0)),
        pl.BlockSpec((_ROWS, _K), lambda i: (i, 0)),
    ],
    out_specs=[
        pl.BlockSpec((_ROWS, _K * _K), lambda i: (i, 0)),
        pl.BlockSpec((_ROWS, _K), lambda i: (i, 0)),
    ],
    out_shape=[
        jax.ShapeDtypeStruct((_N, _K * _K), jnp.float32),
        jax.ShapeDtypeStruct((_N, _K), jnp.float32),
    ],
)


def kernel(tensor, mask_idxs):
    t_flat = tensor.reshape(-1)
    idx_flat = mask_idxs.astype(jnp.int32).reshape(-1)
    valid = mask_idxs >= 0
    neigh = jnp.where(
        valid,
        jnp.take(t_flat, jnp.where(valid, mask_idxs, 0).reshape(-1),
                 axis=0).reshape(_N, _K),
        0.0)
    del idx_flat
    coef_flat, target = _tc_outer(t_flat.reshape(_N, 1), neigh)
    return coef_flat.reshape(_N, _K, _K), target


# speedup vs baseline: 18.1714x; 18.1714x over previous
import functools

import jax
import jax.numpy as jnp
from jax import lax
from jax.experimental import pallas as pl
from jax.experimental.pallas import tpu as pltpu
from jax.experimental.pallas import tpu_sc as plsc

_N = 4096
_K = 32
_LANES = 16
_NC, _NS = 2, 16
_NW = _NC * _NS
_CHUNK = _N * _K // _NW

_ROWS = 512


def _sc_gather_body(t_hbm, idx_hbm, out_hbm, table_v, idx_v, out_v, sem_t, sem_i):
    wid = lax.axis_index("s") * _NC + lax.axis_index("c")
    base = wid * _CHUNK
    cp_t = pltpu.async_copy(t_hbm, table_v, sem_t)
    cp_i = pltpu.async_copy(idx_hbm.at[pl.ds(base, _CHUNK)], idx_v, sem_i)
    cp_t.wait()
    cp_i.wait()

    @pl.loop(0, _CHUNK // _LANES)
    def _(i):
        iv = idx_v[pl.ds(i * _LANES, _LANES)]
        valid = iv >= jnp.zeros((_LANES,), jnp.int32)
        safe = jnp.maximum(iv, jnp.zeros((_LANES,), jnp.int32))
        g = plsc.load_gather(table_v, [safe])
        out_v[pl.ds(i * _LANES, _LANES)] = jnp.where(
            valid, g, jnp.zeros((_LANES,), jnp.float32))

    pltpu.sync_copy(out_v, out_hbm.at[pl.ds(base, _CHUNK)])


@functools.cache
def _sc_gather():
    return pl.kernel(
        _sc_gather_body,
        out_type=jax.ShapeDtypeStruct((_N * _K,), jnp.float32),
        mesh=plsc.VectorSubcoreMesh(
            core_axis_name="c", subcore_axis_name="s",
            num_cores=_NC, num_subcores=_NS),
        scratch_types=[
            pltpu.VMEM((_N,), jnp.float32),
            pltpu.VMEM((_CHUNK,), jnp.int32),
            pltpu.VMEM((_CHUNK,), jnp.float32),
            pltpu.SemaphoreType.DMA,
            pltpu.SemaphoreType.DMA,
        ],
        compiler_params=pltpu.CompilerParams(needs_layout_passes=False),
    )


def _sel_matrices():
    import numpy as np
    c = np.arange(_K * _K)
    m = np.arange(_K)[:, None]
    A = (c[None, :] // _K == m).astype(np.float32)
    B = (c[None, :] % _K == m).astype(np.float32)
    return jnp.asarray(A), jnp.asarray(B)


def _tc_outer_body(t_ref, n_ref, A_ref, B_ref, coef_ref, tgt_ref):
    nb = n_ref[...]
    a = jnp.dot(nb, A_ref[...], preferred_element_type=jnp.float32)
    b = jnp.dot(nb, B_ref[...], preferred_element_type=jnp.float32)
    coef_ref[...] = a * b
    tgt_ref[...] = t_ref[...] * nb


_tc_outer = pl.pallas_call(
    _tc_outer_body,
    grid=(_N // _ROWS,),
    in_specs=[
        pl.BlockSpec((_ROWS, 1), lambda i: (i, 0)),
        pl.BlockSpec((_ROWS, _K), lambda i: (i, 0)),
        pl.BlockSpec((_K, _K * _K), lambda i: (0, 0)),
        pl.BlockSpec((_K, _K * _K), lambda i: (0, 0)),
    ],
    out_specs=[
        pl.BlockSpec((_ROWS, _K * _K), lambda i: (i, 0)),
        pl.BlockSpec((_ROWS, _K), lambda i: (i, 0)),
    ],
    out_shape=[
        jax.ShapeDtypeStruct((_N, _K * _K), jnp.float32),
        jax.ShapeDtypeStruct((_N, _K), jnp.float32),
    ],
)


def kernel(tensor, mask_idxs):
    t_flat = tensor.reshape(-1)
    idx_flat = mask_idxs.astype(jnp.int32).reshape(-1)
    neigh = _sc_gather()(t_flat, idx_flat).reshape(_N, _K)
    A, B = _sel_matrices()
    coef_flat, target = _tc_outer(t_flat.reshape(_N, 1), neigh, A, B)
    return coef_flat.reshape(_N, _K, _K), target
